# baseline (speedup 1.0000x reference)
import functools

import jax
import jax.numpy as jnp
from jax import lax
from jax.experimental import pallas as pl
from jax.experimental.pallas import tpu as pltpu
from jax.experimental.pallas import tpu_sc as plsc

B, N, D = 128, 8192, 64
NC, NS = 2, 16
NW = NC * NS
BPW = B // NW

ZB = 4

_mesh = plsc.VectorSubcoreMesh(
    core_axis_name="c", subcore_axis_name="s", num_cores=NC, num_subcores=NS
)


@functools.partial(
    pl.kernel,
    out_type=jax.ShapeDtypeStruct((B, D, 128), jnp.float32),
    mesh=_mesh,
    scratch_types=[
        pltpu.VMEM((B,), jnp.int32),
        pltpu.VMEM((BPW, D, 128), jnp.float32),
        pltpu.SemaphoreType.DMA,
    ],
    compiler_params=pltpu.CompilerParams(needs_layout_passes=False),
)
def _gather_chunks(x_hbm, tgt_hbm, out_hbm, tgt_v, chunk_v, gsem):
    c = lax.axis_index("c")
    s = lax.axis_index("s")
    w = c * NS + s
    bstart = w * BPW

    pltpu.sync_copy(tgt_hbm, tgt_v)
    lanes = lax.iota(jnp.int32, 16)
    win = (w // 4) * 16
    off = (w % 4) * BPW
    t16 = tgt_v[pl.ds(win, 16)]
    c16 = (t16 >> 7) << 7
    copies = []
    for j in range(BPW):
        c_j = jnp.sum(jnp.where(lanes == off + j, c16, 0))
        c_j = pl.multiple_of(c_j, 128)
        copies.append(
            pltpu.async_copy(
                x_hbm.at[pl.ds((bstart + j) * D, D), pl.ds(c_j, 128)],
                chunk_v.at[j],
                gsem,
            )
        )
    for cpy in copies:
        cpy.wait()
    pltpu.sync_copy(chunk_v, out_hbm.at[pl.ds(bstart, BPW)])


def _place_body(tgt_ref, chunk_ref, tvec_ref, zrc_ref, out_hbm,
                strips_v, zsem0, zsem1, psem):
    zsems = [zsem0, zsem1]
    zcopies = []
    for i in range(B // ZB):
        zcopies.append(
            pltpu.async_copy(
                zrc_ref, out_hbm.at[pl.ds(i * ZB, ZB)], zsems[i % 2]
            )
        )
    tm = jnp.reshape(lax.rem(tvec_ref[...], 128), (B, 1, 1))
    lane = lax.broadcasted_iota(jnp.int32, (B, D, 128), 2)
    strips_v[...] = jnp.where(lane == tm, chunk_ref[...], 0.0)
    for cpy in zcopies:
        cpy.wait()
    pcopies = []
    for b in range(B):
        c0 = pl.multiple_of((tgt_ref[b] >> 7) << 7, 128)
        pcopies.append(
            pltpu.async_copy(
                strips_v.at[b], out_hbm.at[b, :, pl.ds(c0, 128)], psem
            )
        )
    for cpy in pcopies:
        cpy.wait()


_place = pl.pallas_call(
    _place_body,
    grid_spec=pltpu.PrefetchScalarGridSpec(
        num_scalar_prefetch=1,
        grid=(1,),
        in_specs=[
            pl.BlockSpec((B, D, 128), lambda i, tgt: (0, 0, 0)),
            pl.BlockSpec((B, 1), lambda i, tgt: (0, 0)),
            pl.BlockSpec((ZB, D, N), lambda i, tgt: (0, 0, 0)),
        ],
        out_specs=pl.BlockSpec(memory_space=pltpu.MemorySpace.HBM),
        scratch_shapes=[
            pltpu.VMEM((B, D, 128), jnp.float32),
            pltpu.SemaphoreType.DMA,
            pltpu.SemaphoreType.DMA,
            pltpu.SemaphoreType.DMA,
        ],
    ),
    out_shape=jax.ShapeDtypeStruct((B, D, N), jnp.float32),
)


def kernel(input, target):
    x2d = input.transpose(0, 2, 1).reshape(B * D, N)
    tgt = target.astype(jnp.int32)
    chunks = _gather_chunks(x2d, tgt)
    zrc = jnp.zeros((ZB, D, N), jnp.float32)
    out3 = _place(tgt, chunks, tgt[:, None], zrc)
    return out3.transpose(0, 2, 1)


# speedup vs baseline: 1.1014x; 1.1014x over previous
import functools

import jax
import jax.numpy as jnp
from jax import lax
from jax.experimental import pallas as pl
from jax.experimental.pallas import tpu as pltpu
from jax.experimental.pallas import tpu_sc as plsc

B, N, D = 128, 8192, 64
NC, NS = 2, 16
NW = NC * NS
BPW = B // NW

ZB = 2

_mesh = plsc.VectorSubcoreMesh(
    core_axis_name="c", subcore_axis_name="s", num_cores=NC, num_subcores=NS
)


@functools.partial(
    pl.kernel,
    out_type=jax.ShapeDtypeStruct((B, D, 128), jnp.float32),
    mesh=_mesh,
    scratch_types=[
        pltpu.VMEM((B,), jnp.int32),
        pltpu.VMEM((BPW, D, 128), jnp.float32),
        pltpu.SemaphoreType.DMA,
    ],
    compiler_params=pltpu.CompilerParams(needs_layout_passes=False),
)
def _gather_chunks(x_hbm, tgt_hbm, out_hbm, tgt_v, chunk_v, gsem):
    c = lax.axis_index("c")
    s = lax.axis_index("s")
    w = c * NS + s
    bstart = w * BPW

    pltpu.sync_copy(tgt_hbm, tgt_v)
    lanes = lax.iota(jnp.int32, 16)
    win = (w // 4) * 16
    off = (w % 4) * BPW
    t16 = tgt_v[pl.ds(win, 16)]
    c16 = (t16 >> 7) << 7
    copies = []
    for j in range(BPW):
        c_j = jnp.sum(jnp.where(lanes == off + j, c16, 0))
        c_j = pl.multiple_of(c_j, 128)
        copies.append(
            pltpu.async_copy(
                x_hbm.at[pl.ds((bstart + j) * D, D), pl.ds(c_j, 128)],
                chunk_v.at[j],
                gsem,
            )
        )
    for cpy in copies:
        cpy.wait()
    pltpu.sync_copy(chunk_v, out_hbm.at[pl.ds(bstart, BPW)])


def _blast_body(zrc_ref, out_hbm, zsem0, zsem1):
    zsems = [zsem0, zsem1]
    zcopies = []
    for i in range(B // ZB):
        zcopies.append(
            pltpu.async_copy(
                zrc_ref, out_hbm.at[pl.ds(i * ZB, ZB)], zsems[i % 2]
            )
        )
    for cpy in zcopies:
        cpy.wait()


_blast = pl.pallas_call(
    _blast_body,
    grid=(1,),
    in_specs=[pl.BlockSpec((ZB, D, N), lambda i: (0, 0, 0))],
    out_specs=pl.BlockSpec(memory_space=pltpu.MemorySpace.HBM),
    scratch_shapes=[pltpu.SemaphoreType.DMA, pltpu.SemaphoreType.DMA],
    out_shape=jax.ShapeDtypeStruct((B, D, N), jnp.float32),
)


def _patch_body(tgt_ref, zin_hbm, chunk_ref, tvec_ref, out_hbm,
                strips_v, psem):
    del zin_hbm
    tm = jnp.reshape(lax.rem(tvec_ref[...], 128), (B, 1, 1))
    lane = lax.broadcasted_iota(jnp.int32, (B, D, 128), 2)
    strips_v[...] = jnp.where(lane == tm, chunk_ref[...], 0.0)
    pcopies = []
    for b in range(B):
        c0 = pl.multiple_of((tgt_ref[b] >> 7) << 7, 128)
        pcopies.append(
            pltpu.async_copy(
                strips_v.at[b], out_hbm.at[b, :, pl.ds(c0, 128)], psem
            )
        )
    for cpy in pcopies:
        cpy.wait()


_patch = pl.pallas_call(
    _patch_body,
    grid_spec=pltpu.PrefetchScalarGridSpec(
        num_scalar_prefetch=1,
        grid=(1,),
        in_specs=[
            pl.BlockSpec(memory_space=pltpu.MemorySpace.HBM),
            pl.BlockSpec((B, D, 128), lambda i, tgt: (0, 0, 0)),
            pl.BlockSpec((B, 1), lambda i, tgt: (0, 0)),
        ],
        out_specs=pl.BlockSpec(memory_space=pltpu.MemorySpace.HBM),
        scratch_shapes=[
            pltpu.VMEM((B, D, 128), jnp.float32),
            pltpu.SemaphoreType.DMA,
        ],
    ),
    out_shape=jax.ShapeDtypeStruct((B, D, N), jnp.float32),
    input_output_aliases={1: 0},
)


def kernel(input, target):
    x2d = input.transpose(0, 2, 1).reshape(B * D, N)
    tgt = target.astype(jnp.int32)
    chunks = _gather_chunks(x2d, tgt)
    zrc = jnp.zeros((ZB, D, N), jnp.float32)
    zeros3 = _blast(zrc)
    out3 = _patch(tgt, zeros3, chunks, tgt[:, None])
    return out3.transpose(0, 2, 1)
